# Initial kernel scaffold; baseline (speedup 1.0000x reference)
import jax
import jax.numpy as jnp
from jax.experimental import pallas as pl


def kernel(ban_ids, item_ids, aux1, aux2, aux3):
    raise NotImplementedError("write your pallas kernel here")


# speedup vs baseline: 10.9953x; 10.9953x over previous
import functools

import jax
import jax.numpy as jnp
from jax.experimental import pallas as pl
from jax.experimental.pallas import tpu as pltpu

NUM_ITEMS = 200
PAD = 0
NEG = -1000000000.0


def _spop_block(item_ref, ban_ref, out_ref):
    B, S = item_ref.shape
    K = ban_ref.shape[2]
    C = NUM_ITEMS

    iota_i = jax.lax.broadcasted_iota(jnp.int32, (B, C), 1)
    counts = jnp.zeros((B, C), jnp.float32)
    last = jnp.full((B, 1), -1, jnp.int32)
    for s in range(S):
        col = item_ref[:, s].reshape(B, 1)
        valid = col != PAD
        counts = counts + jnp.where((iota_i == col) & valid, 1.0, 0.0)
        last = jnp.where(valid, col, last)
    counts = counts - jnp.where(iota_i == last, 1.0, 0.0)

    m = jnp.max(counts, axis=1, keepdims=True)
    exp_row = jnp.exp(counts - m)

    iota_f = jax.lax.broadcasted_iota(jnp.int32, (B, S, C), 2).astype(
        jnp.float32
    )
    d = ban_ref[:, :, 0].astype(jnp.float32)[:, :, None] - iota_f
    for k in range(1, K):
        d = d * (ban_ref[:, :, k].astype(jnp.float32)[:, :, None] - iota_f)
    banned = d == 0.0

    sum_unb = jnp.sum(
        jnp.where(banned, 0.0, exp_row[:, None, :]), axis=2
    )
    lse = m + jnp.log(sum_unb)
    out_ref[...] = (
        jnp.where(banned, NEG, counts[:, None, :]) - lse[:, :, None]
    )


@functools.partial(jax.jit, static_argnames=("interpret",))
def _spop(ban_ids, item_ids, interpret=False):
    N, S = item_ids.shape
    K = ban_ids.shape[2]
    B = 128
    grid = (N // B,)
    pi = pl.pallas_call(
        _spop_block,
        grid=grid,
        in_specs=[
            pl.BlockSpec((B, S), lambda i: (i, 0)),
            pl.BlockSpec((B, S, K), lambda i: (i, 0, 0)),
        ],
        out_specs=pl.BlockSpec((B, S, NUM_ITEMS), lambda i: (i, 0, 0)),
        out_shape=jax.ShapeDtypeStruct((N, S, NUM_ITEMS), jnp.float32),
        compiler_params=pltpu.CompilerParams(
            dimension_semantics=("parallel",),
        ),
        interpret=interpret,
    )(item_ids, ban_ids)
    return pi


def kernel(ban_ids, item_ids, aux1, aux2, aux3):
    pi = _spop(ban_ids, item_ids)
    n, s = item_ids.shape
    v = jnp.zeros((n, s, 1), jnp.float32)
    return (pi, v)
